# baseline (speedup 1.0000x reference)
import functools

import jax
import jax.numpy as jnp
from jax import lax
from jax.experimental import pallas as pl
from jax.experimental.pallas import tpu as pltpu
from jax.experimental.pallas import tpu_sc as plsc

N_FEATURES = 26
FEATURE_SIZE = 100000
EMB_DIM = 32
BATCH = 16384
HB = BATCH // 2
TW = 100352
QS = 24576
ROUNDS = [24576] * 4 + [2048]
CH = TW // 2
TABLE_COLS = N_FEATURES * FEATURE_SIZE

_INFO = plsc.get_sparse_core_info()
NC = _INFO.num_cores
NS = _INFO.num_subcores


TAIL = TABLE_COLS // 128 * 128
C0MAX = TAIL - TW


@functools.partial(
    pl.kernel,
    mesh=plsc.VectorSubcoreMesh(core_axis_name="c", subcore_axis_name="s"),
    out_type=jax.ShapeDtypeStruct((N_FEATURES, EMB_DIM, BATCH), jnp.float32),
    scratch_types=[
        pltpu.VMEM_SHARED((8, QS), jnp.float32),
        pltpu.VMEM_SHARED((8, BATCH), jnp.float32),
        pltpu.VMEM((CH,), jnp.float32),
        pltpu.VMEM((32, 128), jnp.int32),
        pltpu.VMEM((BATCH,), jnp.float32),
        pltpu.VMEM((BATCH // 4,), jnp.float32),
        pltpu.VMEM((EMB_DIM, 128), jnp.float32),
    ],
    compiler_params=pltpu.CompilerParams(
        use_tc_tiling_on_sc=True, needs_layout_passes=False
    ),
)
def _lookup_kernel(ids_hbm, table_hbm, tail_hbm, out_hbm,
                   sp_tab, sp_out, col_v, ids_v, val_v, mrg_v, tail_v):
    cid = lax.axis_index("c")
    sid = lax.axis_index("s")
    srow = lax.rem(sid, 8)
    half = sid // 8
    rh0 = half * CH
    pltpu.sync_copy(tail_hbm, tail_v)

    def per_feature(t, carry):
        c0 = jnp.minimum(t * FEATURE_SIZE // 128 * 128, C0MAX)
        loff = t * FEATURE_SIZE - c0
        tail_lo = TAIL - t * FEATURE_SIZE

        for g in range(2):
            row0 = pl.multiple_of(cid * 16 + g * 8, 8)
            col = cid * 16 + g * 8 + srow

            q0 = 0
            for rw in ROUNDS:
                pw = rw // 16
                poff = sid * pw
                pltpu.sync_copy(
                    table_hbm.at[pl.ds(row0, 8),
                                 pl.ds(pl.multiple_of(c0 + q0 + poff, 128),
                                       pw)],
                    sp_tab.at[pl.ds(0, 8), pl.ds(pl.multiple_of(poff, 128),
                                                 pw)])
                plsc.subcore_barrier()
                b0 = min(q0 + rw, CH)
                if b0 > q0:
                    @pl.when(half == 0)
                    def _pull0():
                        pltpu.sync_copy(sp_tab.at[srow, pl.ds(0, b0 - q0)],
                                        col_v.at[pl.ds(q0, b0 - q0)])
                a1 = max(q0, CH)
                if q0 + rw > a1:
                    @pl.when(half == 1)
                    def _pull1():
                        pltpu.sync_copy(
                            sp_tab.at[srow, pl.ds(a1 - q0, q0 + rw - a1)],
                            col_v.at[pl.ds(a1 - CH, q0 + rw - a1)])
                plsc.subcore_barrier()
                q0 += rw

            for qq in range(4):
                pltpu.sync_copy(
                    ids_hbm.at[pl.ds(
                        pl.multiple_of(t * 128 + qq * 32, 8), 32)],
                    ids_v)

                def vec(k, inner):
                    sl = pl.ds(qq * (BATCH // 4) + k * 16, 16)
                    ids16 = ids_v[k // 8, pl.ds((k % 8) * 16, 16)]
                    lo = ids16 + loff - rh0
                    m = jnp.logical_and(lo >= 0, lo < CH)
                    v = plsc.load_gather(
                        col_v,
                        [jnp.minimum(jnp.maximum(lo, 0), CH - 1)])
                    v = jnp.where(m, v, 0.0)
                    tloc = jnp.maximum(ids16 - tail_lo, 0)
                    tv = plsc.load_gather(
                        tail_v, [jnp.broadcast_to(col, (16,)), tloc])
                    v = jnp.where(
                        jnp.logical_and(ids16 >= tail_lo, rh0 > 0), tv, v)
                    val_v[sl] = v
                    return inner

                lax.fori_loop(0, BATCH // 64, vec, 0)

            @pl.when(half == 0)
            def _push0():
                pltpu.sync_copy(val_v, sp_out.at[srow])

            plsc.subcore_barrier()

            @pl.when(half == 1)
            def _push1():
                for qq in range(4):
                    pltpu.sync_copy(
                        sp_out.at[srow, pl.ds(qq * (BATCH // 4), BATCH // 4)],
                        mrg_v)

                    def madd(k, inner):
                        sl = pl.ds(qq * (BATCH // 4) + k * 16, 16)
                        val_v[sl] = val_v[sl] + mrg_v[pl.ds(k * 16, 16)]
                        return inner

                    lax.fori_loop(0, BATCH // 64, madd, 0)
                pltpu.sync_copy(val_v, sp_out.at[srow])

            plsc.subcore_barrier()

            @pl.when(sid == 0)
            def _flush():
                pltpu.sync_copy(sp_out, out_hbm.at[t, pl.ds(row0, 8)])

            plsc.subcore_barrier()
        return carry

    lax.fori_loop(0, N_FEATURES, per_feature, 0)


def kernel(feature_ids, embed_weight):
    tail = jnp.pad(embed_weight[TAIL:].T, ((0, 0), (0, 128 - (TABLE_COLS - TAIL))))
    ids3 = feature_ids.T.reshape(N_FEATURES * 128, 128)
    out = _lookup_kernel(ids3, embed_weight.T, tail)
    return out.transpose(2, 0, 1)


# speedup vs baseline: 1.5718x; 1.5718x over previous
import functools

import jax
import jax.numpy as jnp
from jax import lax
from jax.experimental import pallas as pl
from jax.experimental.pallas import tpu as pltpu
from jax.experimental.pallas import tpu_sc as plsc

N_FEATURES = 26
FEATURE_SIZE = 100000
EMB_DIM = 32
BATCH = 16384
HB = BATCH // 2
TW = 100352
QS = 16384
ROUNDS = [16384] * 6 + [2048]
TABLE_COLS = N_FEATURES * FEATURE_SIZE

_INFO = plsc.get_sparse_core_info()
NC = _INFO.num_cores
NS = _INFO.num_subcores


TAIL = TABLE_COLS // 128 * 128
C0MAX = TAIL - TW


@functools.partial(
    pl.kernel,
    mesh=plsc.VectorSubcoreMesh(core_axis_name="c", subcore_axis_name="s"),
    out_type=jax.ShapeDtypeStruct((N_FEATURES, EMB_DIM, BATCH), jnp.float32),
    scratch_types=[
        pltpu.VMEM_SHARED((8, QS), jnp.float32),
        pltpu.VMEM_SHARED((8, HB), jnp.float32),
        pltpu.VMEM((TW,), jnp.float32),
        pltpu.VMEM((32, 128), jnp.int32),
        pltpu.VMEM((HB,), jnp.float32),
        pltpu.VMEM((EMB_DIM, 128), jnp.float32),
    ],
    compiler_params=pltpu.CompilerParams(
        use_tc_tiling_on_sc=True, needs_layout_passes=False
    ),
)
def _lookup_kernel(ids_hbm, table_hbm, tail_hbm, out_hbm,
                   sp_tab, sp_out, col_v, ids_v, val_v, tail_v):
    cid = lax.axis_index("c")
    sid = lax.axis_index("s")
    srow = lax.rem(sid, 8)
    hh0 = (sid // 8) * (HB // 128)
    pltpu.sync_copy(tail_hbm, tail_v)

    def per_feature(t, carry):
        c0 = jnp.minimum(t * FEATURE_SIZE // 128 * 128, C0MAX)
        loff = t * FEATURE_SIZE - c0
        tail_lo = TAIL - t * FEATURE_SIZE

        for g in range(2):
            row0 = pl.multiple_of(cid * 16 + g * 8, 8)
            col = cid * 16 + g * 8 + srow

            q0 = 0
            for rw in ROUNDS:
                pw = rw // 16
                poff = sid * pw
                pltpu.sync_copy(
                    table_hbm.at[pl.ds(row0, 8),
                                 pl.ds(pl.multiple_of(c0 + q0 + poff, 128),
                                       pw)],
                    sp_tab.at[pl.ds(0, 8), pl.ds(pl.multiple_of(poff, 128),
                                                 pw)])
                plsc.subcore_barrier()
                pltpu.sync_copy(sp_tab.at[srow, pl.ds(0, rw)],
                                col_v.at[pl.ds(q0, rw)])
                plsc.subcore_barrier()
                q0 += rw

            for qq in range(2):
                pltpu.sync_copy(
                    ids_hbm.at[pl.ds(
                        pl.multiple_of(t * 128 + hh0 + qq * 32, 8), 32)],
                    ids_v)

                def vec(k, inner):
                    sl = pl.ds(qq * (HB // 2) + k * 16, 16)
                    ids16 = ids_v[k // 8, pl.ds((k % 8) * 16, 16)]
                    v = plsc.load_gather(
                        col_v, [jnp.minimum(ids16 + loff, TW - 1)])
                    tloc = jnp.maximum(ids16 - tail_lo, 0)
                    tv = plsc.load_gather(
                        tail_v, [jnp.broadcast_to(col, (16,)), tloc])
                    v = jnp.where(ids16 >= tail_lo, tv, v)
                    val_v[sl] = v
                    return inner

                lax.fori_loop(0, HB // 32, vec, 0)
            for hh in range(2):
                @pl.when(sid // 8 == hh)
                def _push():
                    pltpu.sync_copy(val_v, sp_out.at[srow])

                plsc.subcore_barrier()

                @pl.when(sid == 0)
                def _flush():
                    pltpu.sync_copy(
                        sp_out,
                        out_hbm.at[t, pl.ds(row0, 8), pl.ds(hh * HB, HB)])

                plsc.subcore_barrier()
        return carry

    lax.fori_loop(0, N_FEATURES, per_feature, 0)


def kernel(feature_ids, embed_weight):
    tail = jnp.pad(embed_weight[TAIL:].T, ((0, 0), (0, 128 - (TABLE_COLS - TAIL))))
    ids3 = feature_ids.T.reshape(N_FEATURES * 128, 128)
    out = _lookup_kernel(ids3, embed_weight.T, tail)
    return out.transpose(2, 0, 1)
